# baseline (speedup 1.0000x reference)
import functools

import jax
import jax.numpy as jnp
import numpy as np
from jax.experimental import pallas as pl
from jax.experimental.pallas import tpu as pltpu
from jax.experimental.shard_map import shard_map
from jax.sharding import Mesh, PartitionSpec as P


def _ghost_body(x_ref, wp0_ref, wp1_ref, wp2_ref,
                wc0_ref, wc1_ref, wc2_ref, wf_ref, b_ref,
                o_ref, t_ref, *, H, W, init_ch, inter, d_list):
    HW = H * W
    t_ref[3 * inter:, :] = x_ref[0]
    col = jax.lax.broadcasted_iota(jnp.int32, (init_ch, HW), 1) % W
    wps = (wp0_ref, wp1_ref, wp2_ref)
    wcs = (wc0_ref, wc1_ref, wc2_ref)
    for i, d in enumerate(d_list):
        base = (2 - i) * inter
        src = t_ref[base + inter:, :]
        s = jnp.dot(wps[i][...], src, preferred_element_type=jnp.float32)
        s = jnp.maximum(s, 0.0)
        t_ref[base:base + init_ch, :] = s
        rz = jnp.zeros((init_ch, W * d), jnp.float32)
        pr = jnp.concatenate([rz, s, rz], axis=1)
        stk = jnp.concatenate(
            [pr[:, 0:HW], s, pr[:, 2 * W * d:2 * W * d + HW]], axis=0)
        z = jnp.dot(wcs[i][...], stk, preferred_element_type=jnp.float32)
        co = z.shape[0] // 3
        cz = jnp.zeros((co, d), jnp.float32)
        z0 = jnp.concatenate([cz, z[0:co]], axis=1)[:, 0:HW]
        z2 = jnp.concatenate([z[2 * co:], cz], axis=1)[:, d:d + HW]
        x2 = z[co:2 * co]
        x2 = x2 + jnp.where(col[:co] >= d, z0, 0.0)
        x2 = x2 + jnp.where(col[:co] < W - d, z2, 0.0)
        x2 = jnp.maximum(x2, 0.0)
        t_ref[base + init_ch:base + inter, :] = x2[:inter - init_ch]
    y = jnp.dot(wf_ref[...], t_ref[...], preferred_element_type=jnp.float32)
    y = y + b_ref[...]
    o_ref[0] = y.astype(o_ref.dtype)


def _ghost_forward(x, w_primary_0, w_primary_1, w_primary_2,
                   w_cheap_0, w_cheap_1, w_cheap_2, w_final, b_final):
    B, cin, H, W = x.shape
    HW = H * W
    init_ch = w_primary_0.shape[0]
    new_ch = w_cheap_0.shape[0]
    C = w_final.shape[1]
    inter = (C - cin) // 3
    d_list = (1, 2, 3)

    xf = x.reshape(B, cin, HW)
    def _flat(w):
        return jnp.concatenate(
            [w[:, :, :, kw].transpose(0, 2, 1).reshape(new_ch, 3 * init_ch)
             for kw in range(3)], axis=0)

    body = functools.partial(_ghost_body, H=H, W=W, init_ch=init_ch,
                             inter=inter, d_list=d_list)
    const = lambda b: (0, 0)
    out = pl.pallas_call(
        body,
        out_shape=jax.ShapeDtypeStruct((B, cin, HW), x.dtype),
        grid=(B,),
        in_specs=[
            pl.BlockSpec((1, cin, HW), lambda b: (b, 0, 0)),
            pl.BlockSpec((init_ch, cin), const),
            pl.BlockSpec((init_ch, cin + inter), const),
            pl.BlockSpec((init_ch, cin + 2 * inter), const),
            pl.BlockSpec((3 * new_ch, 3 * init_ch), const),
            pl.BlockSpec((3 * new_ch, 3 * init_ch), const),
            pl.BlockSpec((3 * new_ch, 3 * init_ch), const),
            pl.BlockSpec((cin, C), const),
            pl.BlockSpec((cin, 1), const),
        ],
        out_specs=pl.BlockSpec((1, cin, HW), lambda b: (b, 0, 0)),
        scratch_shapes=[
            pltpu.VMEM((C, HW), jnp.float32),
        ],
        compiler_params=pltpu.CompilerParams(
            dimension_semantics=("parallel",)),
    )(xf, w_primary_0, w_primary_1, w_primary_2,
      _flat(w_cheap_0), _flat(w_cheap_1), _flat(w_cheap_2),
      w_final, b_final.reshape(cin, 1))
    return out.reshape(B, cin, H, W)


def kernel(x, w_primary_0, w_primary_1, w_primary_2,
           w_cheap_0, w_cheap_1, w_cheap_2, w_final, b_final):
    args = (x, w_primary_0, w_primary_1, w_primary_2,
            w_cheap_0, w_cheap_1, w_cheap_2, w_final, b_final)
    devs = jax.devices()
    if len(devs) >= 2 and x.shape[0] % 2 == 0:
        mesh = Mesh(np.array(devs[:2]), ("b",))
        fwd = shard_map(
            _ghost_forward, mesh=mesh,
            in_specs=(P("b"),) + (P(),) * 8,
            out_specs=P("b"), check_rep=False)
        return fwd(*args)
    return _ghost_forward(*args)


# speedup vs baseline: 3.5318x; 3.5318x over previous
import functools

import jax
import jax.numpy as jnp
from jax.experimental import pallas as pl
from jax.experimental.pallas import tpu as pltpu


def _ghost_body(x_ref, wp0_ref, wp1_ref, wp2_ref,
                wc0_ref, wc1_ref, wc2_ref, wf_ref, b_ref,
                o_ref, t_ref, sp_ref, *, H, W, init_ch, inter, d_list):
    HW = H * W
    m0 = W * max(d_list)
    t_ref[3 * inter:, :] = x_ref[0]
    sp_ref[:, 0:m0] = jnp.zeros((init_ch, m0), sp_ref.dtype)
    sp_ref[:, m0 + HW:] = jnp.zeros((init_ch, m0), sp_ref.dtype)
    col = jax.lax.broadcasted_iota(jnp.int32, (init_ch, HW), 1) % W
    wps = (wp0_ref, wp1_ref, wp2_ref)
    wcs = (wc0_ref, wc1_ref, wc2_ref)
    for i, d in enumerate(d_list):
        base = (2 - i) * inter
        src = t_ref[base + inter:, :]
        s32 = jnp.dot(wps[i][...], src, preferred_element_type=jnp.float32)
        s = jnp.maximum(s32, 0.0).astype(t_ref.dtype)
        t_ref[base:base + init_ch, :] = s
        sp_ref[:, m0:m0 + HW] = s
        wc = wcs[i]
        z = jnp.dot(wc[:, 0:init_ch], sp_ref[:, m0 - W * d:m0 - W * d + HW],
                    preferred_element_type=jnp.float32)
        z = z + jnp.dot(wc[:, init_ch:2 * init_ch], sp_ref[:, m0:m0 + HW],
                        preferred_element_type=jnp.float32)
        z = z + jnp.dot(wc[:, 2 * init_ch:],
                        sp_ref[:, m0 + W * d:m0 + W * d + HW],
                        preferred_element_type=jnp.float32)
        co = z.shape[0] // 3
        cz = jnp.zeros((co, d), jnp.float32)
        z0 = jnp.concatenate([cz, z[0:co]], axis=1)[:, 0:HW]
        z2 = jnp.concatenate([z[2 * co:], cz], axis=1)[:, d:d + HW]
        x2 = z[co:2 * co]
        x2 = x2 + jnp.where(col[:co] >= d, z0, 0.0)
        x2 = x2 + jnp.where(col[:co] < W - d, z2, 0.0)
        x2 = jnp.maximum(x2, 0.0).astype(t_ref.dtype)
        t_ref[base + init_ch:base + inter, :] = x2[:inter - init_ch]
    y = jnp.dot(wf_ref[...], t_ref[...], preferred_element_type=jnp.float32)
    y = y + b_ref[...]
    o_ref[0] = y.astype(o_ref.dtype)


def _ghost_forward(x, w_primary_0, w_primary_1, w_primary_2,
                   w_cheap_0, w_cheap_1, w_cheap_2, w_final, b_final):
    B, cin, H, W = x.shape
    HW = H * W
    init_ch = w_primary_0.shape[0]
    new_ch = w_cheap_0.shape[0]
    C = w_final.shape[1]
    inter = (C - cin) // 3
    d_list = (1, 2, 3)
    bf16 = jnp.bfloat16

    xf = x.astype(bf16).reshape(B, cin, HW)
    def _flat(w):
        return jnp.concatenate(
            [w[:, :, :, kw].transpose(0, 2, 1).reshape(new_ch, 3 * init_ch)
             for kw in range(3)], axis=0).astype(bf16)

    body = functools.partial(_ghost_body, H=H, W=W, init_ch=init_ch,
                             inter=inter, d_list=d_list)
    const = lambda b: (0, 0)
    out = pl.pallas_call(
        body,
        out_shape=jax.ShapeDtypeStruct((B, cin, HW), x.dtype),
        grid=(B,),
        in_specs=[
            pl.BlockSpec((1, cin, HW), lambda b: (b, 0, 0)),
            pl.BlockSpec((init_ch, cin), const),
            pl.BlockSpec((init_ch, cin + inter), const),
            pl.BlockSpec((init_ch, cin + 2 * inter), const),
            pl.BlockSpec((3 * new_ch, 3 * init_ch), const),
            pl.BlockSpec((3 * new_ch, 3 * init_ch), const),
            pl.BlockSpec((3 * new_ch, 3 * init_ch), const),
            pl.BlockSpec((cin, C), const),
            pl.BlockSpec((cin, 1), const),
        ],
        out_specs=pl.BlockSpec((1, cin, HW), lambda b: (b, 0, 0)),
        scratch_shapes=[
            pltpu.VMEM((C, HW), bf16),
            pltpu.VMEM((init_ch, HW + 2 * W * max(d_list)), bf16),
        ],
        compiler_params=pltpu.CompilerParams(
            dimension_semantics=("parallel",)),
    )(xf, w_primary_0.astype(bf16), w_primary_1.astype(bf16),
      w_primary_2.astype(bf16),
      _flat(w_cheap_0), _flat(w_cheap_1), _flat(w_cheap_2),
      w_final.astype(bf16), b_final.reshape(cin, 1))
    return out.reshape(B, cin, H, W)


def kernel(x, w_primary_0, w_primary_1, w_primary_2,
           w_cheap_0, w_cheap_1, w_cheap_2, w_final, b_final):
    return _ghost_forward(x, w_primary_0, w_primary_1, w_primary_2,
                          w_cheap_0, w_cheap_1, w_cheap_2, w_final, b_final)


# speedup vs baseline: 3.7970x; 1.0751x over previous
import functools

import jax
import jax.numpy as jnp
from jax.experimental import pallas as pl
from jax.experimental.pallas import tpu as pltpu


def _ghost_body(x_ref, wp0_ref, wp1_ref, wp2_ref,
                wc0_ref, wc1_ref, wc2_ref, wf_ref, b_ref,
                o_ref, t_ref, sp_ref, *, H, W, init_ch, inter, d_list):
    HW = H * W
    m0 = W * max(d_list)
    t_ref[3 * inter:, :] = x_ref[0]
    sp_ref[:, 0:m0] = jnp.zeros((init_ch, m0), sp_ref.dtype)
    sp_ref[:, m0 + HW:] = jnp.zeros((init_ch, m0), sp_ref.dtype)
    col = jax.lax.broadcasted_iota(jnp.int32, (init_ch, HW), 1) % W
    wps = (wp0_ref, wp1_ref, wp2_ref)
    wcs = (wc0_ref, wc1_ref, wc2_ref)
    for i, d in enumerate(d_list):
        base = (2 - i) * inter
        src = t_ref[base + inter:, :]
        s32 = jnp.dot(wps[i][...], src, preferred_element_type=jnp.float32)
        s = jnp.maximum(s32, 0.0).astype(t_ref.dtype)
        t_ref[base:base + init_ch, :] = s
        sp_ref[:, m0:m0 + HW] = s
        wc = wcs[i]
        z = jnp.dot(wc[:, 0:init_ch], sp_ref[:, m0 - W * d:m0 - W * d + HW],
                    preferred_element_type=jnp.float32)
        z = z + jnp.dot(wc[:, init_ch:2 * init_ch], sp_ref[:, m0:m0 + HW],
                        preferred_element_type=jnp.float32)
        z = z + jnp.dot(wc[:, 2 * init_ch:],
                        sp_ref[:, m0 + W * d:m0 + W * d + HW],
                        preferred_element_type=jnp.float32)
        co = z.shape[0] // 3
        cz = jnp.zeros((co, d), jnp.float32)
        z0 = jnp.concatenate([cz, z[0:co]], axis=1)[:, 0:HW]
        z2 = jnp.concatenate([z[2 * co:], cz], axis=1)[:, d:d + HW]
        x2 = z[co:2 * co]
        x2 = x2 + jnp.where(col[:co] >= d, z0, 0.0)
        x2 = x2 + jnp.where(col[:co] < W - d, z2, 0.0)
        x2 = jnp.maximum(x2, 0.0).astype(t_ref.dtype)
        t_ref[base + init_ch:base + inter, :] = x2[:inter - init_ch]
    y = jnp.dot(wf_ref[...], t_ref[...], preferred_element_type=jnp.float32)
    y = y + b_ref[...]
    o_ref[0] = y.astype(o_ref.dtype)


def _ghost_forward(x, w_primary_0, w_primary_1, w_primary_2,
                   w_cheap_0, w_cheap_1, w_cheap_2, w_final, b_final):
    B, cin, H, W = x.shape
    HW = H * W
    init_ch = w_primary_0.shape[0]
    new_ch = w_cheap_0.shape[0]
    C = w_final.shape[1]
    inter = (C - cin) // 3
    d_list = (1, 2, 3)

    xf = x.reshape(B, cin, HW)
    def _flat(w):
        return jnp.concatenate(
            [w[:, :, :, kw].transpose(0, 2, 1).reshape(new_ch, 3 * init_ch)
             for kw in range(3)], axis=0)

    body = functools.partial(_ghost_body, H=H, W=W, init_ch=init_ch,
                             inter=inter, d_list=d_list)
    const = lambda b: (0, 0)
    out = pl.pallas_call(
        body,
        out_shape=jax.ShapeDtypeStruct((B, cin, HW), x.dtype),
        grid=(B,),
        in_specs=[
            pl.BlockSpec((1, cin, HW), lambda b: (b, 0, 0)),
            pl.BlockSpec((init_ch, cin), const),
            pl.BlockSpec((init_ch, cin + inter), const),
            pl.BlockSpec((init_ch, cin + 2 * inter), const),
            pl.BlockSpec((3 * new_ch, 3 * init_ch), const),
            pl.BlockSpec((3 * new_ch, 3 * init_ch), const),
            pl.BlockSpec((3 * new_ch, 3 * init_ch), const),
            pl.BlockSpec((cin, C), const),
            pl.BlockSpec((cin, 1), const),
        ],
        out_specs=pl.BlockSpec((1, cin, HW), lambda b: (b, 0, 0)),
        scratch_shapes=[
            pltpu.VMEM((C, HW), jnp.float32),
            pltpu.VMEM((init_ch, HW + 2 * W * max(d_list)), jnp.float32),
        ],
        compiler_params=pltpu.CompilerParams(
            dimension_semantics=("parallel",)),
    )(xf, w_primary_0, w_primary_1, w_primary_2,
      _flat(w_cheap_0), _flat(w_cheap_1), _flat(w_cheap_2),
      w_final, b_final.reshape(cin, 1))
    return out.reshape(B, cin, H, W)


def kernel(x, w_primary_0, w_primary_1, w_primary_2,
           w_cheap_0, w_cheap_1, w_cheap_2, w_final, b_final):
    return _ghost_forward(x, w_primary_0, w_primary_1, w_primary_2,
                          w_cheap_0, w_cheap_1, w_cheap_2, w_final, b_final)


# speedup vs baseline: 4.6529x; 1.2254x over previous
import functools

import jax
import jax.numpy as jnp
from jax.experimental import pallas as pl
from jax.experimental.pallas import tpu as pltpu


def _ghost_body(x_ref, wp0_ref, wp1_ref, wp2_ref,
                wc0_ref, wc1_ref, wc2_ref, wf_ref, b_ref,
                o_ref, t_ref, *, H, W, init_ch, inter, d_list):
    HW = H * W
    n = len(d_list)
    col = jax.lax.broadcasted_iota(jnp.int32, (init_ch, HW), 1) % W
    wps = (wp0_ref, wp1_ref, wp2_ref)
    wcs = (wc0_ref, wc1_ref, wc2_ref)
    for i, d in enumerate(d_list):
        base = (n - 1 - i) * inter
        kt = i * inter
        s = jnp.dot(wps[i][:, kt:], x_ref[0],
                    preferred_element_type=jnp.float32)
        if kt:
            s = s + jnp.dot(wps[i][:, :kt], t_ref[base + inter:, :],
                            preferred_element_type=jnp.float32)
        s = jnp.maximum(s, 0.0)
        t_ref[base:base + init_ch, :] = s
        rz = jnp.zeros((init_ch, W * d), jnp.float32)
        pr = jnp.concatenate([rz, s, rz], axis=1)
        stk = jnp.concatenate(
            [pr[:, 0:HW], s, pr[:, 2 * W * d:2 * W * d + HW]], axis=0)
        z = jnp.dot(wcs[i][...], stk, preferred_element_type=jnp.float32)
        co = z.shape[0] // 3
        cz = jnp.zeros((co, d), jnp.float32)
        z0 = jnp.concatenate([cz, z[0:co]], axis=1)[:, 0:HW]
        z2 = jnp.concatenate([z[2 * co:], cz], axis=1)[:, d:d + HW]
        x2 = z[co:2 * co]
        x2 = x2 + jnp.where(col[:co] >= d, z0, 0.0)
        x2 = x2 + jnp.where(col[:co] < W - d, z2, 0.0)
        x2 = jnp.maximum(x2, 0.0)
        t_ref[base + init_ch:base + inter, :] = x2[:inter - init_ch]
    y = jnp.dot(wf_ref[:, :n * inter], t_ref[...],
                preferred_element_type=jnp.float32)
    y = y + jnp.dot(wf_ref[:, n * inter:], x_ref[0],
                    preferred_element_type=jnp.float32)
    y = y + b_ref[...]
    o_ref[0] = y.astype(o_ref.dtype)


def _ghost_forward(x, w_primary_0, w_primary_1, w_primary_2,
                   w_cheap_0, w_cheap_1, w_cheap_2, w_final, b_final):
    B, cin, H, W = x.shape
    HW = H * W
    init_ch = w_primary_0.shape[0]
    new_ch = w_cheap_0.shape[0]
    C = w_final.shape[1]
    inter = (C - cin) // 3
    d_list = (1, 2, 3)

    xf = x.reshape(B, cin, HW)
    def _flat(w):
        return jnp.concatenate(
            [w[:, :, :, kw].transpose(0, 2, 1).reshape(new_ch, 3 * init_ch)
             for kw in range(3)], axis=0)

    body = functools.partial(_ghost_body, H=H, W=W, init_ch=init_ch,
                             inter=inter, d_list=d_list)
    const = lambda b: (0, 0)
    out = pl.pallas_call(
        body,
        out_shape=jax.ShapeDtypeStruct((B, cin, HW), x.dtype),
        grid=(B,),
        in_specs=[
            pl.BlockSpec((1, cin, HW), lambda b: (b, 0, 0)),
            pl.BlockSpec((init_ch, cin), const),
            pl.BlockSpec((init_ch, cin + inter), const),
            pl.BlockSpec((init_ch, cin + 2 * inter), const),
            pl.BlockSpec((3 * new_ch, 3 * init_ch), const),
            pl.BlockSpec((3 * new_ch, 3 * init_ch), const),
            pl.BlockSpec((3 * new_ch, 3 * init_ch), const),
            pl.BlockSpec((cin, C), const),
            pl.BlockSpec((cin, 1), const),
        ],
        out_specs=pl.BlockSpec((1, cin, HW), lambda b: (b, 0, 0)),
        scratch_shapes=[
            pltpu.VMEM((3 * inter, HW), jnp.float32),
        ],
        compiler_params=pltpu.CompilerParams(
            dimension_semantics=("parallel",)),
    )(xf, w_primary_0, w_primary_1, w_primary_2,
      _flat(w_cheap_0), _flat(w_cheap_1), _flat(w_cheap_2),
      w_final, b_final.reshape(cin, 1))
    return out.reshape(B, cin, H, W)


def kernel(x, w_primary_0, w_primary_1, w_primary_2,
           w_cheap_0, w_cheap_1, w_cheap_2, w_final, b_final):
    return _ghost_forward(x, w_primary_0, w_primary_1, w_primary_2,
                          w_cheap_0, w_cheap_1, w_cheap_2, w_final, b_final)


# speedup vs baseline: 4.7644x; 1.0239x over previous
import functools

import jax
import jax.numpy as jnp
from jax.experimental import pallas as pl
from jax.experimental.pallas import tpu as pltpu


def _ghost_body(x_ref, wp0_ref, wp1_ref, wp2_ref,
                wc0_ref, wc1_ref, wc2_ref, wf_ref, b_ref,
                o_ref, t_ref, *, H, W, init_ch, inter, d_list):
    HW = H * W
    t_ref[3 * inter:, :] = x_ref[0]
    col = jax.lax.broadcasted_iota(jnp.int32, (init_ch, HW), 1) % W
    wps = (wp0_ref, wp1_ref, wp2_ref)
    wcs = (wc0_ref, wc1_ref, wc2_ref)
    for i, d in enumerate(d_list):
        base = (2 - i) * inter
        src = t_ref[base + inter:, :]
        s = jnp.dot(wps[i][...], src, preferred_element_type=jnp.float32)
        s = jnp.maximum(s, 0.0)
        t_ref[base:base + init_ch, :] = s
        rz = jnp.zeros((init_ch, W * d), jnp.float32)
        pr = jnp.concatenate([rz, s, rz], axis=1)
        stk = jnp.concatenate(
            [pr[:, 0:HW], s, pr[:, 2 * W * d:2 * W * d + HW]], axis=0)
        z = jnp.dot(wcs[i][...], stk, preferred_element_type=jnp.float32)
        co = z.shape[0] // 3
        cz = jnp.zeros((co, d), jnp.float32)
        z0 = jnp.concatenate([cz, z[0:co]], axis=1)[:, 0:HW]
        z2 = jnp.concatenate([z[2 * co:], cz], axis=1)[:, d:d + HW]
        x2 = z[co:2 * co]
        x2 = x2 + jnp.where(col[:co] >= d, z0, 0.0)
        x2 = x2 + jnp.where(col[:co] < W - d, z2, 0.0)
        x2 = jnp.maximum(x2, 0.0)
        t_ref[base + init_ch:base + inter, :] = x2[:inter - init_ch]
    y = jnp.dot(wf_ref[...], t_ref[...], preferred_element_type=jnp.float32)
    y = y + b_ref[...]
    o_ref[0] = y.astype(o_ref.dtype)


def _ghost_forward(x, w_primary_0, w_primary_1, w_primary_2,
                   w_cheap_0, w_cheap_1, w_cheap_2, w_final, b_final):
    B, cin, H, W = x.shape
    HW = H * W
    init_ch = w_primary_0.shape[0]
    new_ch = w_cheap_0.shape[0]
    C = w_final.shape[1]
    inter = (C - cin) // 3
    d_list = (1, 2, 3)

    xf = x.reshape(B, cin, HW)
    def _flat(w):
        return jnp.concatenate(
            [w[:, :, :, kw].transpose(0, 2, 1).reshape(new_ch, 3 * init_ch)
             for kw in range(3)], axis=0)

    body = functools.partial(_ghost_body, H=H, W=W, init_ch=init_ch,
                             inter=inter, d_list=d_list)
    const = lambda b: (0, 0)
    out = pl.pallas_call(
        body,
        out_shape=jax.ShapeDtypeStruct((B, cin, HW), x.dtype),
        grid=(B,),
        in_specs=[
            pl.BlockSpec((1, cin, HW), lambda b: (b, 0, 0)),
            pl.BlockSpec((init_ch, cin), const),
            pl.BlockSpec((init_ch, cin + inter), const),
            pl.BlockSpec((init_ch, cin + 2 * inter), const),
            pl.BlockSpec((3 * new_ch, 3 * init_ch), const),
            pl.BlockSpec((3 * new_ch, 3 * init_ch), const),
            pl.BlockSpec((3 * new_ch, 3 * init_ch), const),
            pl.BlockSpec((cin, C), const),
            pl.BlockSpec((cin, 1), const),
        ],
        out_specs=pl.BlockSpec((1, cin, HW), lambda b: (b, 0, 0)),
        scratch_shapes=[
            pltpu.VMEM((C, HW), jnp.float32),
        ],
        compiler_params=pltpu.CompilerParams(
            dimension_semantics=("parallel",)),
    )(xf, w_primary_0, w_primary_1, w_primary_2,
      _flat(w_cheap_0), _flat(w_cheap_1), _flat(w_cheap_2),
      w_final, b_final.reshape(cin, 1))
    return out.reshape(B, cin, H, W)


def kernel(x, w_primary_0, w_primary_1, w_primary_2,
           w_cheap_0, w_cheap_1, w_cheap_2, w_final, b_final):
    return _ghost_forward(x, w_primary_0, w_primary_1, w_primary_2,
                          w_cheap_0, w_cheap_1, w_cheap_2, w_final, b_final)
